# baseline (speedup 1.0000x reference)
import jax
import jax.numpy as jnp
from jax.experimental import pallas as pl
from jax.experimental.pallas import tpu as pltpu

B = 128
I = 512
O = 512
W = 512
K = 154
WB = 8
NC = 2


def _route_body(x_ref, gw_ref, gb_ref, dwt_ref):
    logits = jax.lax.dot_general(
        gw_ref[...], x_ref[...], (((1,), (1,)), ((), ())),
        preferred_element_type=jnp.float32,
    ) + gb_ref[...]

    bits = jax.lax.bitcast_convert_type(logits, jnp.uint32)
    sign = bits >> jnp.uint32(31)
    key = jnp.where(sign == jnp.uint32(1), ~bits, bits | jnp.uint32(0x80000000))

    lo = jnp.zeros((1, B), jnp.uint32)
    hi = jnp.full((1, B), jnp.uint32(0xFFFFFFFF))

    def body(_, carry):
        lo, hi = carry
        mid = lo + ((hi - lo) >> jnp.uint32(1))
        cnt = jnp.sum((key >= mid).astype(jnp.int32), axis=0, keepdims=True)
        ge = cnt >= K
        return jnp.where(ge, mid, lo), jnp.where(ge, hi, mid)

    lo, hi = jax.lax.fori_loop(0, 32, body, (lo, hi))

    mask = key >= lo
    m = jnp.max(logits, axis=0, keepdims=True)
    e = jnp.where(mask, jnp.exp(logits - m), 0.0)
    denom = jnp.sum(e, axis=0, keepdims=True)
    dwt_ref[...] = e / denom


def _moe_body(x_ref, dwt_ref0, dwt_ref1, tiles_ref0, tiles_ref1, outt_ref):
    i = pl.program_id(0)

    @pl.when(i == 0)
    def _init():
        outt_ref[...] = jnp.zeros_like(outt_ref)

    x = x_ref[...]
    acc = outt_ref[...]
    for dwt_r, tiles_r in ((dwt_ref0, tiles_ref0), (dwt_ref1, tiles_ref1)):
        dwb = dwt_r[0]
        for j in range(WB):
            t = tiles_r[j]
            pt = jax.lax.dot_general(
                t, x, (((1,), (1,)), ((), ())), preferred_element_type=jnp.float32
            )
            acc = acc + dwb[j : j + 1, :] * pt
    outt_ref[...] = acc


def kernel(x, gate_w, gate_b, tiles):
    gb2 = jnp.broadcast_to(gate_b[:, None], (W, B))

    dwt = pl.pallas_call(
        _route_body,
        in_specs=[
            pl.BlockSpec((B, I), lambda: (0, 0)),
            pl.BlockSpec((W, I), lambda: (0, 0)),
            pl.BlockSpec((W, B), lambda: (0, 0)),
        ],
        out_specs=pl.BlockSpec((W, B), lambda: (0, 0)),
        out_shape=jax.ShapeDtypeStruct((W, B), jnp.float32),
    )(x, gate_w, gb2)

    dwt3 = dwt.reshape(W // WB, WB, B)

    half = W // (2 * WB)
    outt = pl.pallas_call(
        _moe_body,
        grid=(half,),
        in_specs=[
            pl.BlockSpec((B, I), lambda i: (0, 0)),
            pl.BlockSpec((1, WB, B), lambda i: (i, 0, 0)),
            pl.BlockSpec((1, WB, B), lambda i: (half + i, 0, 0)),
            pl.BlockSpec((WB, O, I), lambda i: (i, 0, 0)),
            pl.BlockSpec((WB, O, I), lambda i: (half + i, 0, 0)),
        ],
        out_specs=pl.BlockSpec((O, B), lambda i: (0, 0)),
        out_shape=jax.ShapeDtypeStruct((O, B), jnp.float32),
    )(x, dwt3, dwt3, tiles, tiles)
    return outt.T


# speedup vs baseline: 1.0320x; 1.0320x over previous
import jax
import jax.numpy as jnp
from jax.experimental import pallas as pl
from jax.experimental.pallas import tpu as pltpu

B = 128
I = 512
O = 512
W = 512
K = 154
WB = 8
GRID = W // WB


def _fused_body(x_ref, gw_ref, gb_ref, tiles_ref, out_ref, dwt_scr, acc_scr):
    i = pl.program_id(0)

    @pl.when(i == 0)
    def _route():
        logits = jax.lax.dot_general(
            gw_ref[...], x_ref[...], (((1,), (1,)), ((), ())),
            preferred_element_type=jnp.float32,
        ) + gb_ref[...]

        bits = jax.lax.bitcast_convert_type(logits, jnp.uint32)
        sign = bits >> jnp.uint32(31)
        key = jnp.where(
            sign == jnp.uint32(1), ~bits, bits | jnp.uint32(0x80000000)
        )

        lo = jnp.zeros((1, B), jnp.uint32)
        hi = jnp.full((1, B), jnp.uint32(0xFFFFFFFF))

        def body(_, carry):
            lo, hi = carry
            mid = lo + ((hi - lo) >> jnp.uint32(1))
            cnt = jnp.sum((key >= mid).astype(jnp.int32), axis=0, keepdims=True)
            ge = cnt >= K
            return jnp.where(ge, mid, lo), jnp.where(ge, hi, mid)

        lo, hi = jax.lax.fori_loop(0, 32, body, (lo, hi))

        mask = key >= lo
        m = jnp.max(logits, axis=0, keepdims=True)
        e = jnp.where(mask, jnp.exp(logits - m), 0.0)
        denom = jnp.sum(e, axis=0, keepdims=True)
        dwt_scr[...] = e / denom
        acc_scr[...] = jnp.zeros_like(acc_scr)

    @pl.when(i > 0)
    def _accum():
        x = x_ref[...]
        blk = i - 1
        dwb = dwt_scr[pl.ds(blk * WB, WB), :]
        acc = acc_scr[...]
        for j in range(WB):
            t = tiles_ref[j]
            pt = jax.lax.dot_general(
                t, x, (((1,), (1,)), ((), ())),
                preferred_element_type=jnp.float32,
            )
            acc = acc + dwb[j : j + 1, :] * pt
        acc_scr[...] = acc

    @pl.when(i == GRID)
    def _final():
        out_ref[...] = acc_scr[...].T


def kernel(x, gate_w, gate_b, tiles):
    gb2 = jnp.broadcast_to(gate_b[:, None], (W, B))

    out = pl.pallas_call(
        _fused_body,
        grid=(GRID + 1,),
        in_specs=[
            pl.BlockSpec((B, I), lambda i: (0, 0)),
            pl.BlockSpec((W, I), lambda i: (0, 0)),
            pl.BlockSpec((W, B), lambda i: (0, 0)),
            pl.BlockSpec(
                (WB, O, I), lambda i: (jnp.maximum(i - 1, 0), 0, 0)
            ),
        ],
        out_specs=pl.BlockSpec((B, O), lambda i: (0, 0)),
        out_shape=jax.ShapeDtypeStruct((B, O), jnp.float32),
        scratch_shapes=[
            pltpu.VMEM((W, B), jnp.float32),
            pltpu.VMEM((O, B), jnp.float32),
        ],
    )(x, gate_w, gb2, tiles)
    return out
